# Initial kernel scaffold; baseline (speedup 1.0000x reference)
import jax
import jax.numpy as jnp
from jax.experimental import pallas as pl


def kernel(x, edge_index, edge_weights, W, b):
    raise NotImplementedError("write your pallas kernel here")


# speedup vs baseline: 3.6828x; 3.6828x over previous
import functools

import jax
import jax.numpy as jnp
from jax import lax
from jax.experimental import pallas as pl
from jax.experimental.pallas import tpu as pltpu
from jax.experimental.pallas import tpu_sc as plsc

N_NODES = 10000
D = 128
E = 320000
NC = 2
NS = 16
NW = NC * NS
CHUNK = 128
NCHUNK = 80
EPW = NCHUNK * CHUNK
E_PAD = NW * EPW
N_PAD = 10240
ZROWS = 128
ROWS_PER_TILE = N_PAD // NS
ZCOPIES = ROWS_PER_TILE // ZROWS


def _sc_message_passing(x, src, dst, w):
    mesh = plsc.VectorSubcoreMesh(
        core_axis_name="c", subcore_axis_name="s", num_cores=NC, num_subcores=NS
    )

    @functools.partial(
        pl.kernel,
        out_type=jax.ShapeDtypeStruct((NC, N_PAD, D), jnp.float32),
        mesh=mesh,
        scratch_types=[
            pltpu.VMEM((NCHUNK, CHUNK), jnp.int32),
            pltpu.VMEM((NCHUNK, CHUNK), jnp.int32),
            pltpu.VMEM((NCHUNK, CHUNK), jnp.float32),
            pltpu.VMEM((CHUNK, D), jnp.float32),
            pltpu.VMEM_SHARED((N_PAD, D), jnp.float32),
            pltpu.SemaphoreType.DMA,
        ],
    )
    def k(x_hbm, src_hbm, dst_hbm, w_hbm, out_hbm,
          src_v, dst_v, w_v, rows_v, h_sh, sem):
        c = lax.axis_index("c")
        s = lax.axis_index("s")
        wid = s * NC + c

        pltpu.sync_copy(src_hbm.at[wid], src_v)
        pltpu.sync_copy(dst_hbm.at[wid], dst_v)
        pltpu.sync_copy(w_hbm.at[wid], w_v)

        zeros = jnp.zeros((16,), jnp.float32)

        def zrow(i, carry):
            for g in range(D // 16):
                rows_v[i, pl.ds(g * 16, 16)] = zeros
            return carry

        lax.fori_loop(0, ZROWS, zrow, 0)
        row0 = s * ROWS_PER_TILE
        for t in range(ZCOPIES):
            pltpu.sync_copy(rows_v, h_sh.at[pl.ds(row0 + t * ZROWS, ZROWS)])
        plsc.subcore_barrier()

        def chunk_body(j, carry):
            pltpu.async_copy(x_hbm.at[src_v.at[j]], rows_v, sem).wait()

            def edge16_body(t, c2):
                wvec = w_v[j, pl.ds(t * 16, 16)]
                for i in range(16):
                    wval = wvec[i]
                    e = t * 16 + i
                    for g in range(D // 16):
                        sl = pl.ds(g * 16, 16)
                        rows_v[e, sl] = rows_v[e, sl] * wval
                return c2

            lax.fori_loop(0, CHUNK // 16, edge16_body, 0)
            pltpu.sync_copy(rows_v, h_sh.at[dst_v.at[j]], add=True)
            return carry

        lax.fori_loop(0, NCHUNK, chunk_body, 0)
        plsc.subcore_barrier()

        for t in range(ZCOPIES):
            r = row0 + t * ZROWS
            pltpu.sync_copy(h_sh.at[pl.ds(r, ZROWS)],
                            out_hbm.at[c, pl.ds(r, ZROWS)])

    return k(x, src, dst, w)


def _tc_linear(h0, h1, wt, b2):
    blk = 1000

    def body(h0_ref, h1_ref, wt_ref, b_ref, o_ref):
        hsum = h0_ref[...] + h1_ref[...]
        o_ref[...] = (
            jnp.dot(hsum, wt_ref[...], preferred_element_type=jnp.float32)
            + b_ref[...]
        )

    return pl.pallas_call(
        body,
        grid=(N_NODES // blk,),
        in_specs=[
            pl.BlockSpec((blk, D), lambda i: (i, 0)),
            pl.BlockSpec((blk, D), lambda i: (i, 0)),
            pl.BlockSpec((D, D), lambda i: (0, 0)),
            pl.BlockSpec((1, D), lambda i: (0, 0)),
        ],
        out_specs=pl.BlockSpec((blk, D), lambda i: (i, 0)),
        out_shape=jax.ShapeDtypeStruct((N_NODES, D), jnp.float32),
    )(h0, h1, wt, b2)


def kernel(x, edge_index, edge_weights, W, b):
    pad = E_PAD - E
    src = jnp.concatenate(
        [edge_index[0].astype(jnp.int32), jnp.zeros((pad,), jnp.int32)]
    ).reshape(NW, NCHUNK, CHUNK)
    dst = jnp.concatenate(
        [edge_index[1].astype(jnp.int32), jnp.zeros((pad,), jnp.int32)]
    ).reshape(NW, NCHUNK, CHUNK)
    w = jnp.concatenate(
        [edge_weights.reshape(E).astype(jnp.float32),
         jnp.zeros((pad,), jnp.float32)]
    ).reshape(NW, NCHUNK, CHUNK)
    h2 = _sc_message_passing(x, src, dst, w)
    return _tc_linear(h2[0], h2[1], W.T, b.reshape(1, D))


# speedup vs baseline: 4.3832x; 1.1902x over previous
import functools

import jax
import jax.numpy as jnp
from jax import lax
from jax.experimental import pallas as pl
from jax.experimental.pallas import tpu as pltpu
from jax.experimental.pallas import tpu_sc as plsc

N_NODES = 10000
D = 128
E = 320000
NC = 2
NS = 16
NW = NC * NS
CHUNK = 32
NCHUNK = 320
EPW = NCHUNK * CHUNK
E_PAD = NW * EPW
SROWS = EPW // 128
N_PAD = 10240
ZROWS = CHUNK
ROWS_PER_TILE = N_PAD // NS
ZCOPIES = ROWS_PER_TILE // ZROWS


def _sc_message_passing(x, sd, w):
    mesh = plsc.VectorSubcoreMesh(
        core_axis_name="c", subcore_axis_name="s", num_cores=NC, num_subcores=NS
    )

    @functools.partial(
        pl.kernel,
        out_type=jax.ShapeDtypeStruct((NC, N_PAD, D), jnp.float32),
        mesh=mesh,
        scratch_types=[
            pltpu.VMEM((SROWS, 128), jnp.int32),
            pltpu.VMEM((SROWS, 128), jnp.float32),
            pltpu.VMEM((2, CHUNK), jnp.int32),
            pltpu.VMEM((2, CHUNK), jnp.int32),
            pltpu.VMEM((CHUNK, D), jnp.float32),
            pltpu.VMEM((CHUNK, D), jnp.float32),
            pltpu.VMEM((CHUNK, D), jnp.float32),
            pltpu.VMEM((CHUNK, D), jnp.float32),
            pltpu.VMEM_SHARED((N_PAD, D), jnp.float32),
            pltpu.SemaphoreType.DMA,
            pltpu.SemaphoreType.DMA,
            pltpu.SemaphoreType.DMA,
            pltpu.SemaphoreType.DMA,
        ],
    )
    def k(x_hbm, sd_hbm, w_hbm, out_hbm,
          sd_v, w_v, sidx, didx, gbuf0, gbuf1, sbuf0, sbuf1, h_sh,
          gsem0, gsem1, ssem0, ssem1):
        c = lax.axis_index("c")
        s = lax.axis_index("s")
        wid = s * NC + c
        gbuf = (gbuf0, gbuf1)
        sbuf = (sbuf0, sbuf1)
        gsem = (gsem0, gsem1)
        ssem = (ssem0, ssem1)

        pltpu.sync_copy(sd_hbm.at[wid], sd_v)
        pltpu.sync_copy(w_hbm.at[wid], w_v)

        def unpack_src(j_, b_):
            for t in range(CHUNK // 16):
                ssl = pl.ds((j_ & 3) * CHUNK + t * 16, 16)
                sidx[b_, pl.ds(t * 16, 16)] = sd_v[j_ >> 2, ssl] & 0xFFFF

        def unpack_dst(j_, b_):
            for t in range(CHUNK // 16):
                ssl = pl.ds((j_ & 3) * CHUNK + t * 16, 16)
                didx[b_, pl.ds(t * 16, 16)] = sd_v[j_ >> 2, ssl] >> 16

        zeros = jnp.zeros((16,), jnp.float32)

        def zrow(i, carry):
            for g in range(D // 16):
                sbuf0[i, pl.ds(g * 16, 16)] = zeros
            return carry

        lax.fori_loop(0, CHUNK, zrow, 0)
        row0 = s * ROWS_PER_TILE
        for t in range(ZCOPIES):
            pltpu.sync_copy(sbuf0, h_sh.at[pl.ds(row0 + t * ZROWS, ZROWS)])
        plsc.subcore_barrier()

        unpack_src(jnp.int32(0), 0)
        unpack_src(jnp.int32(1), 1)
        pltpu.async_copy(x_hbm.at[sidx.at[0]], gbuf0, gsem0)
        pltpu.async_copy(x_hbm.at[sidx.at[1]], gbuf1, gsem1)

        def pair_body(jj, carry):
            j0 = jj * 2
            for b in range(2):
                j = j0 + b
                gb, sb = gbuf[b], sbuf[b]
                pltpu.make_async_copy(x_hbm.at[sidx.at[b]], gb,
                                      gsem[b]).wait()
                @pl.when(j >= 2)
                def _():
                    pltpu.make_async_copy(
                        sb, h_sh.at[didx.at[b]], ssem[b]).wait()

                unpack_dst(j, b)
                unpack_src(lax.min(j + 2, NCHUNK - 1), b)

                for t in range(CHUNK // 16):
                    wvec = w_v[j >> 2, pl.ds((j & 3) * CHUNK + t * 16, 16)]
                    for i in range(16):
                        wval = wvec[i]
                        e = t * 16 + i
                        for g in range(D // 16):
                            sl = pl.ds(g * 16, 16)
                            sb[e, sl] = gb[e, sl] * wval

                @pl.when(j + 2 < NCHUNK)
                def _():
                    pltpu.async_copy(x_hbm.at[sidx.at[b]], gb, gsem[b])

                pltpu.async_copy(sb, h_sh.at[didx.at[b]], ssem[b], add=True)
            return carry

        lax.fori_loop(0, NCHUNK // 2, pair_body, 0)
        for b in range(2):
            pltpu.make_async_copy(sbuf[b], h_sh.at[didx.at[b]],
                                  ssem[b]).wait()
        plsc.subcore_barrier()

        for t in range(ZCOPIES):
            r = row0 + t * ZROWS
            pltpu.sync_copy(h_sh.at[pl.ds(r, ZROWS)],
                            out_hbm.at[c, pl.ds(r, ZROWS)])

    return k(x, sd, w)


def _tc_linear(h0, h1, wt, b2):
    blk = 1000

    def body(h0_ref, h1_ref, wt_ref, b_ref, o_ref):
        hsum = h0_ref[...] + h1_ref[...]
        o_ref[...] = (
            jnp.dot(hsum, wt_ref[...], preferred_element_type=jnp.float32)
            + b_ref[...]
        )

    return pl.pallas_call(
        body,
        grid=(N_NODES // blk,),
        in_specs=[
            pl.BlockSpec((blk, D), lambda i: (i, 0)),
            pl.BlockSpec((blk, D), lambda i: (i, 0)),
            pl.BlockSpec((D, D), lambda i: (0, 0)),
            pl.BlockSpec((1, D), lambda i: (0, 0)),
        ],
        out_specs=pl.BlockSpec((blk, D), lambda i: (i, 0)),
        out_shape=jax.ShapeDtypeStruct((N_NODES, D), jnp.float32),
    )(h0, h1, wt, b2)


def kernel(x, edge_index, edge_weights, W, b):
    pad = E_PAD - E
    src = edge_index[0].astype(jnp.int32)
    dst = edge_index[1].astype(jnp.int32)
    sd = jnp.concatenate(
        [src + dst * 65536, jnp.zeros((pad,), jnp.int32)]
    ).reshape(NW, SROWS, 128)
    w = jnp.concatenate(
        [edge_weights.reshape(E).astype(jnp.float32),
         jnp.zeros((pad,), jnp.float32)]
    ).reshape(NW, SROWS, 128)
    h2 = _sc_message_passing(x, sd, w)
    return _tc_linear(h2[0], h2[1], W.T, b.reshape(1, D))
